# baseline (speedup 1.0000x reference)
import functools

import jax
import jax.numpy as jnp
from jax.experimental import pallas as pl
from jax.experimental.pallas import tpu as pltpu

_N_Q = 4
_N_E = 1024
_E_DIM = 128
_BETA = 0.25
_BM = 512


def _rvq_kernel(z_ref, w_ref, zq_ref, idx_ref, loss_ref, r_scratch):
    i = pl.program_id(0)
    k = pl.program_id(1)
    w = w_ref[0]

    @pl.when(k == 0)
    def _load():
        r_scratch[...] = z_ref[...]

    r = r_scratch[...]
    wsq = jnp.sum(w * w, axis=1)
    rsq = jnp.sum(r * r, axis=1, keepdims=True)
    mm = jax.lax.dot_general(r, w, (((1,), (1,)), ((), ())),
                             preferred_element_type=jnp.float32)
    score = rsq - 2.0 * mm + wsq[None, :]
    col_iota = jax.lax.broadcasted_iota(jnp.int32, (_BM, _N_E), 1)
    mn = jnp.min(score, axis=1, keepdims=True)
    idx = jnp.min(jnp.where(score <= mn, col_iota, _N_E), axis=1)
    onehot = (col_iota == idx[:, None]).astype(jnp.float32)
    q = jax.lax.dot_general(
        onehot, w, (((1,), (0,)), ((), ())),
        precision=jax.lax.Precision.HIGHEST,
        preferred_element_type=jnp.float32)
    r = r - q
    r_scratch[...] = r
    idx_ref[0, 0, :] = idx

    @pl.when(k == 0)
    def _zq_init():
        zq_ref[...] = q

    @pl.when(k > 0)
    def _zq_acc():
        zq_ref[...] += q

    @pl.when(k == _N_Q - 1)
    def _zq_fin():
        z0 = z_ref[...]
        zq_ref[...] = z0 + (zq_ref[...] - z0)

    loss_blk = jnp.reshape(jnp.sum(r * r), (1, 1))

    @pl.when(jnp.logical_and(i == 0, k == 0))
    def _loss_init():
        loss_ref[...] = jnp.zeros((1, 1), jnp.float32)
    loss_ref[...] += loss_blk


def kernel(z, W0, W1, W2, W3):
    B, T, D = z.shape
    M = B * T
    zf = z.reshape(M, D)
    Ws = jnp.stack([W0, W1, W2, W3], axis=0)
    nblk = M // _BM
    grid = (nblk, _N_Q)
    zq, idx, loss = pl.pallas_call(
        _rvq_kernel,
        grid=grid,
        in_specs=[
            pl.BlockSpec((_BM, _E_DIM), lambda i, k: (i, 0)),
            pl.BlockSpec((1, _N_E, _E_DIM), lambda i, k: (k, 0, 0)),
        ],
        out_specs=[
            pl.BlockSpec((_BM, _E_DIM), lambda i, k: (i, 0)),
            pl.BlockSpec((1, 1, _BM), lambda i, k: (k * (18432 // _BM) + i, 0, 0)),
            pl.BlockSpec((1, 1), lambda i, k: (0, 0)),
        ],
        out_shape=[
            jax.ShapeDtypeStruct((M, D), jnp.float32),
            jax.ShapeDtypeStruct((_N_Q * nblk, 1, _BM), jnp.int32),
            jax.ShapeDtypeStruct((1, 1), jnp.float32),
        ],
        scratch_shapes=[pltpu.VMEM((_BM, _E_DIM), jnp.float32)],
    )(zf, Ws)
    total_loss = loss[0, 0] * ((1.0 + _BETA) / (_N_Q * M * D))
    inds = idx.reshape(_N_Q, M).T.reshape(B, T, _N_Q)
    return (zq.reshape(B, T, D), total_loss, inds)


# speedup vs baseline: 1.3028x; 1.3028x over previous
import functools

import jax
import jax.numpy as jnp
from jax.experimental import pallas as pl
from jax.experimental.pallas import tpu as pltpu

_N_Q = 4
_N_E = 1024
_E_DIM = 128
_BETA = 0.25
_BM = 512


def _rvq_kernel(z_ref, w_ref, zq_ref, idx_ref, loss_ref, r_scratch):
    i = pl.program_id(0)
    k = pl.program_id(1)
    w = w_ref[0]

    @pl.when(k == 0)
    def _load():
        r_scratch[...] = z_ref[...]

    r = r_scratch[...]
    wsq = jnp.sum(w * w, axis=1)
    rsq = jnp.sum(r * r, axis=1, keepdims=True)
    mm = jax.lax.dot_general(r, w, (((1,), (1,)), ((), ())),
                             preferred_element_type=jnp.float32)
    score = rsq - 2.0 * mm + wsq[None, :]
    col_iota = jax.lax.broadcasted_iota(jnp.int32, (_BM, _N_E), 1)
    mn = jnp.min(score, axis=1, keepdims=True)
    idx = jnp.min(jnp.where(score <= mn, col_iota, _N_E), axis=1)
    onehot = (col_iota == idx[:, None]).astype(jnp.bfloat16)
    w0 = w.astype(jnp.bfloat16)
    w1 = (w - w0.astype(jnp.float32)).astype(jnp.bfloat16)
    w2 = (w - w0.astype(jnp.float32) - w1.astype(jnp.float32)).astype(jnp.bfloat16)

    def _g(wp):
        return jax.lax.dot_general(onehot, wp, (((1,), (0,)), ((), ())),
                                   preferred_element_type=jnp.float32)

    q = (_g(w0) + _g(w1)) + _g(w2)
    r = r - q
    r_scratch[...] = r
    idx_ref[0, 0, :] = idx

    @pl.when(k == 0)
    def _zq_init():
        zq_ref[...] = q

    @pl.when(k > 0)
    def _zq_acc():
        zq_ref[...] += q

    @pl.when(k == _N_Q - 1)
    def _zq_fin():
        z0 = z_ref[...]
        zq_ref[...] = z0 + (zq_ref[...] - z0)

    loss_blk = jnp.reshape(jnp.sum(r * r), (1, 1))

    @pl.when(jnp.logical_and(i == 0, k == 0))
    def _loss_init():
        loss_ref[...] = jnp.zeros((1, 1), jnp.float32)
    loss_ref[...] += loss_blk


def kernel(z, W0, W1, W2, W3):
    B, T, D = z.shape
    M = B * T
    zf = z.reshape(M, D)
    Ws = jnp.stack([W0, W1, W2, W3], axis=0)
    nblk = M // _BM
    grid = (nblk, _N_Q)
    zq, idx, loss = pl.pallas_call(
        _rvq_kernel,
        grid=grid,
        in_specs=[
            pl.BlockSpec((_BM, _E_DIM), lambda i, k: (i, 0)),
            pl.BlockSpec((1, _N_E, _E_DIM), lambda i, k: (k, 0, 0)),
        ],
        out_specs=[
            pl.BlockSpec((_BM, _E_DIM), lambda i, k: (i, 0)),
            pl.BlockSpec((1, 1, _BM), lambda i, k: (k * (18432 // _BM) + i, 0, 0)),
            pl.BlockSpec((1, 1), lambda i, k: (0, 0)),
        ],
        out_shape=[
            jax.ShapeDtypeStruct((M, D), jnp.float32),
            jax.ShapeDtypeStruct((_N_Q * nblk, 1, _BM), jnp.int32),
            jax.ShapeDtypeStruct((1, 1), jnp.float32),
        ],
        scratch_shapes=[pltpu.VMEM((_BM, _E_DIM), jnp.float32)],
    )(zf, Ws)
    total_loss = loss[0, 0] * ((1.0 + _BETA) / (_N_Q * M * D))
    inds = idx.reshape(_N_Q, M).T.reshape(B, T, _N_Q)
    return (zq.reshape(B, T, D), total_loss, inds)


# speedup vs baseline: 1.5167x; 1.1641x over previous
import jax
import jax.numpy as jnp
from jax.experimental import pallas as pl
from jax.experimental.pallas import tpu as pltpu

_N_Q = 4
_N_E = 1024
_E_DIM = 128
_BETA = 0.25
_BM = 1024


def _rvq_kernel(z_ref, w_ref, zq_ref, idx_ref, loss_ref,
                r_scratch, wsq_ref, ws_ref):
    i = pl.program_id(0)
    k = pl.program_id(1)

    @pl.when(i == 0)
    def _stage_consts():
        w = w_ref[0]
        wsq_ref[k, :] = jnp.sum(w * w, axis=1)
        w0 = w.astype(jnp.bfloat16)
        w1 = (w - w0.astype(jnp.float32)).astype(jnp.bfloat16)
        w2 = (w - w0.astype(jnp.float32) - w1.astype(jnp.float32)
              ).astype(jnp.bfloat16)
        ws_ref[k, 0] = w0
        ws_ref[k, 1] = w1
        ws_ref[k, 2] = w2

    @pl.when(k == 0)
    def _load():
        r_scratch[...] = z_ref[...]

    r = r_scratch[...]
    rsq = jnp.sum(r * r, axis=1, keepdims=True)
    mm = jax.lax.dot_general(r * (-2.0), w_ref[0], (((1,), (1,)), ((), ())),
                             preferred_element_type=jnp.float32)
    score = rsq + mm + wsq_ref[k, :][None, :]
    col_iota = jax.lax.broadcasted_iota(jnp.int32, (_BM, _N_E), 1)
    mn = jnp.min(score, axis=1, keepdims=True)
    idx = jnp.min(jnp.where(score <= mn, col_iota, _N_E), axis=1)
    onehot = (col_iota == idx[:, None]).astype(jnp.bfloat16)

    def _g(wp):
        return jax.lax.dot_general(onehot, wp, (((1,), (0,)), ((), ())),
                                   preferred_element_type=jnp.float32)

    q = (_g(ws_ref[k, 0]) + _g(ws_ref[k, 1])) + _g(ws_ref[k, 2])
    r = r - q
    r_scratch[...] = r
    idx_ref[0, 0, :] = idx

    @pl.when(k == 0)
    def _zq_init():
        zq_ref[...] = q

    @pl.when(k > 0)
    def _zq_acc():
        zq_ref[...] += q

    @pl.when(k == _N_Q - 1)
    def _zq_fin():
        z0 = z_ref[...]
        zq_ref[...] = z0 + (zq_ref[...] - z0)

    loss_blk = jnp.reshape(jnp.sum(r * r), (1, 1))

    @pl.when(jnp.logical_and(i == 0, k == 0))
    def _loss_init():
        loss_ref[...] = jnp.zeros((1, 1), jnp.float32)
    loss_ref[...] += loss_blk


def kernel(z, W0, W1, W2, W3):
    B, T, D = z.shape
    M = B * T
    zf = z.reshape(M, D)
    Ws = jnp.stack([W0, W1, W2, W3], axis=0)
    nblk = M // _BM
    grid = (nblk, _N_Q)
    zq, idx, loss = pl.pallas_call(
        _rvq_kernel,
        grid=grid,
        in_specs=[
            pl.BlockSpec((_BM, _E_DIM), lambda i, k: (i, 0)),
            pl.BlockSpec((1, _N_E, _E_DIM), lambda i, k: (k, 0, 0)),
        ],
        out_specs=[
            pl.BlockSpec((_BM, _E_DIM), lambda i, k: (i, 0)),
            pl.BlockSpec((1, 1, _BM), lambda i, k: (k * (18432 // _BM) + i, 0, 0)),
            pl.BlockSpec((1, 1), lambda i, k: (0, 0)),
        ],
        out_shape=[
            jax.ShapeDtypeStruct((M, D), jnp.float32),
            jax.ShapeDtypeStruct((_N_Q * nblk, 1, _BM), jnp.int32),
            jax.ShapeDtypeStruct((1, 1), jnp.float32),
        ],
        scratch_shapes=[
            pltpu.VMEM((_BM, _E_DIM), jnp.float32),
            pltpu.VMEM((_N_Q, _N_E), jnp.float32),
            pltpu.VMEM((_N_Q, 3, _N_E, _E_DIM), jnp.bfloat16),
        ],
    )(zf, Ws)
    total_loss = loss[0, 0] * ((1.0 + _BETA) / (_N_Q * M * D))
    inds = idx.reshape(_N_Q, M).T.reshape(B, T, _N_Q)
    return (zq.reshape(B, T, D), total_loss, inds)


# speedup vs baseline: 1.6053x; 1.0584x over previous
import jax
import jax.numpy as jnp
from jax.experimental import pallas as pl
from jax.experimental.pallas import tpu as pltpu

_N_Q = 4
_N_E = 1024
_E_DIM = 128
_BETA = 0.25
_BM = 2048


def _rvq_kernel(z_ref, w_ref, zq_ref, idx_ref, loss_ref,
                r_scratch, wsq_ref, ws_ref):
    i = pl.program_id(0)
    k = pl.program_id(1)

    @pl.when(i == 0)
    def _stage_consts():
        w = w_ref[0]
        wsq_ref[k, :] = jnp.sum(w * w, axis=1)
        w0 = w.astype(jnp.bfloat16)
        w1 = (w - w0.astype(jnp.float32)).astype(jnp.bfloat16)
        w2 = (w - w0.astype(jnp.float32) - w1.astype(jnp.float32)
              ).astype(jnp.bfloat16)
        ws_ref[k, 0] = w0
        ws_ref[k, 1] = w1
        ws_ref[k, 2] = w2

    @pl.when(k == 0)
    def _load():
        r_scratch[...] = z_ref[...]

    r = r_scratch[...]
    rsq = jnp.sum(r * r, axis=1, keepdims=True)
    mm = jax.lax.dot_general(r * (-2.0), w_ref[0], (((1,), (1,)), ((), ())),
                             preferred_element_type=jnp.float32)
    score = rsq + mm + wsq_ref[k, :][None, :]
    col_iota = jax.lax.broadcasted_iota(jnp.int32, (_BM, _N_E), 1)
    mn = jnp.min(score, axis=1, keepdims=True)
    idx = jnp.min(jnp.where(score <= mn, col_iota, _N_E), axis=1)
    onehot = (col_iota == idx[:, None]).astype(jnp.bfloat16)

    def _g(wp):
        return jax.lax.dot_general(onehot, wp, (((1,), (0,)), ((), ())),
                                   preferred_element_type=jnp.float32)

    q = (_g(ws_ref[k, 0]) + _g(ws_ref[k, 1])) + _g(ws_ref[k, 2])
    r = r - q
    r_scratch[...] = r
    idx_ref[0, 0, :] = idx

    @pl.when(k == 0)
    def _zq_init():
        zq_ref[...] = q

    @pl.when(k > 0)
    def _zq_acc():
        zq_ref[...] += q

    @pl.when(k == _N_Q - 1)
    def _zq_fin():
        z0 = z_ref[...]
        zq_ref[...] = z0 + (zq_ref[...] - z0)

    loss_blk = jnp.reshape(jnp.sum(r * r), (1, 1))

    @pl.when(jnp.logical_and(i == 0, k == 0))
    def _loss_init():
        loss_ref[...] = jnp.zeros((1, 1), jnp.float32)
    loss_ref[...] += loss_blk


def kernel(z, W0, W1, W2, W3):
    B, T, D = z.shape
    M = B * T
    zf = z.reshape(M, D)
    Ws = jnp.stack([W0, W1, W2, W3], axis=0)
    nblk = M // _BM
    grid = (nblk, _N_Q)
    zq, idx, loss = pl.pallas_call(
        _rvq_kernel,
        grid=grid,
        in_specs=[
            pl.BlockSpec((_BM, _E_DIM), lambda i, k: (i, 0)),
            pl.BlockSpec((1, _N_E, _E_DIM), lambda i, k: (k, 0, 0)),
        ],
        out_specs=[
            pl.BlockSpec((_BM, _E_DIM), lambda i, k: (i, 0)),
            pl.BlockSpec((1, 1, _BM), lambda i, k: (k * (18432 // _BM) + i, 0, 0)),
            pl.BlockSpec((1, 1), lambda i, k: (0, 0)),
        ],
        out_shape=[
            jax.ShapeDtypeStruct((M, D), jnp.float32),
            jax.ShapeDtypeStruct((_N_Q * nblk, 1, _BM), jnp.int32),
            jax.ShapeDtypeStruct((1, 1), jnp.float32),
        ],
        scratch_shapes=[
            pltpu.VMEM((_BM, _E_DIM), jnp.float32),
            pltpu.VMEM((_N_Q, _N_E), jnp.float32),
            pltpu.VMEM((_N_Q, 3, _N_E, _E_DIM), jnp.bfloat16),
        ],
    )(zf, Ws)
    total_loss = loss[0, 0] * ((1.0 + _BETA) / (_N_Q * M * D))
    inds = idx.reshape(_N_Q, M).T.reshape(B, T, _N_Q)
    return (zq.reshape(B, T, D), total_loss, inds)
